# baseline (speedup 1.0000x reference)
import functools

import jax
import jax.numpy as jnp
from jax import lax
from jax.experimental import pallas as pl
from jax.experimental.pallas import tpu as pltpu
from jax.experimental.pallas import tpu_sc as plsc

N = 10000
E = 320000
D = 128
NC = 2
NS = 16
L = 16
NW = NC * NS
K = 128
G = 16
NG0 = 3
NG1 = 7
NG = NG0 + NG1
CW = G * NG // 2
TOTG = NS * NG
EPAD = TOTG * G * K
NPAD = 10240
RPS = NPAD // NS

_mesh = plsc.VectorSubcoreMesh(core_axis_name="c", subcore_axis_name="s")
_sc_params = pltpu.CompilerParams(needs_layout_passes=False)


@functools.partial(
    pl.kernel,
    out_type=jax.ShapeDtypeStruct((NC, NPAD), jnp.float32),
    mesh=_mesh,
    compiler_params=_sc_params,
    scratch_types=[
        pltpu.VMEM((CW, K), jnp.int32),
        pltpu.VMEM((CW, K), jnp.float32),
        pltpu.VMEM_SHARED((NPAD,), jnp.float32),
    ],
)
def _sc_deg(dst_hbm, ew_hbm, zdeg_hbm, deg_out, dst_v, ew_v, deg_sh):
    c = lax.axis_index("c")
    s = lax.axis_index("s")
    wid = s * NC + c
    pltpu.sync_copy(zdeg_hbm.at[pl.ds(s * RPS, RPS)],
                    deg_sh.at[pl.ds(s * RPS, RPS)])
    pltpu.sync_copy(dst_hbm.at[wid], dst_v)
    pltpu.sync_copy(ew_hbm.at[wid], ew_v)
    plsc.subcore_barrier()

    @pl.loop(0, CW)
    def _(t):
        pltpu.sync_copy(ew_v.at[t], deg_sh.at[dst_v.at[t]], add=True)

    plsc.subcore_barrier()

    @pl.when(s == 0)
    def _():
        pltpu.sync_copy(deg_sh, deg_out.at[c])


@functools.partial(
    pl.kernel,
    out_type=jax.ShapeDtypeStruct((NC, NPAD, D), jnp.float32),
    mesh=_mesh,
    compiler_params=_sc_params,
    scratch_types=[
        pltpu.VMEM((G, K), jnp.int32),
        pltpu.VMEM((G, K), jnp.int32),
        pltpu.VMEM((G * K,), jnp.float32),
        pltpu.VMEM((K, D), jnp.float32),
        pltpu.VMEM((K, D), jnp.float32),
        pltpu.VMEM_SHARED((NPAD, D), jnp.float32),
        pltpu.SemaphoreType.DMA,
        pltpu.SemaphoreType.DMA,
    ],
)
def _sc_edge(g_hbm, src_hbm, dst_hbm, ew_hbm, zrow_hbm, acc_out,
             src_v, dst_v, ew_v, rows_a, rows_b, acc_sh, sem_a, sem_b):
    c = lax.axis_index("c")
    s = lax.axis_index("s")
    ng = jnp.where(c == 0, NG0, NG1)
    gb = jnp.where(c == 0, s * NG0, NS * NG0 + s * NG1)
    pltpu.sync_copy(zrow_hbm.at[pl.ds(s * RPS, RPS)],
                    acc_sh.at[pl.ds(s * RPS, RPS)])
    plsc.subcore_barrier()

    def start_gather(t, rows, sem):
        pltpu.make_async_copy(g_hbm.at[src_v.at[t]], rows, sem).start()

    def wait_gather(t, rows, sem):
        pltpu.make_async_copy(g_hbm.at[src_v.at[t]], rows, sem).wait()

    def scale(t, rows):
        @pl.loop(0, K)
        def _(k):
            w = plsc.load_gather(ew_v, [jnp.full((L,), t * K + k, jnp.int32)])

            @pl.loop(0, D, step=L)
            def _(j):
                rows[k, pl.ds(j, L)] = rows[k, pl.ds(j, L)] * w

    def scatter_add(t, rows):
        pltpu.sync_copy(rows, acc_sh.at[dst_v.at[t]], add=True)

    @pl.loop(0, NG)
    def _(gi):
      @pl.when(gi < ng)
      def _():
        pltpu.sync_copy(src_hbm.at[gb + gi], src_v)
        pltpu.sync_copy(dst_hbm.at[gb + gi], dst_v)
        pltpu.sync_copy(ew_hbm.at[gb + gi], ew_v)
        start_gather(0, rows_a, sem_a)

        @pl.loop(0, G // 2)
        def _(u):
            ta = 2 * u
            tb = 2 * u + 1
            start_gather(tb, rows_b, sem_b)
            wait_gather(ta, rows_a, sem_a)
            scale(ta, rows_a)
            scatter_add(ta, rows_a)

            @pl.when(u + 1 < G // 2)
            def _():
                start_gather(ta + 2, rows_a, sem_a)

            wait_gather(tb, rows_b, sem_b)
            scale(tb, rows_b)
            scatter_add(tb, rows_b)

    plsc.subcore_barrier()
    pltpu.sync_copy(acc_sh.at[pl.ds(s * RPS, RPS)],
                    acc_out.at[c, pl.ds(s * RPS, RPS)])


def _tc_dinv(deg2):
    def body(deg_ref, out_ref):
        d = deg_ref[0:1, :] + deg_ref[1:2, :] + 1.0
        out_ref[...] = jnp.where(d > 0, lax.rsqrt(d), 0.0)

    return pl.pallas_call(
        body, out_shape=jax.ShapeDtypeStruct((1, NPAD), jnp.float32))(deg2)


def _tc_mm_scale(x, w, dinv):
    def body(x_ref, w_ref, dinv_ref, o_ref):
        h = jnp.dot(x_ref[...], w_ref[...], preferred_element_type=jnp.float32)
        o_ref[...] = h * dinv_ref[...]

    return pl.pallas_call(
        body, out_shape=jax.ShapeDtypeStruct((N, D), jnp.float32))(x, w, dinv)


def _tc_post_mm(acc, g, dinv, b, w):
    def body(acc_ref, g_ref, dinv_ref, b_ref, w_ref, o_ref):
        agg = acc_ref[0, :N, :] + acc_ref[1, :N, :] + g_ref[...]
        x1 = jnp.maximum(dinv_ref[...] * agg + b_ref[...], 0.0)
        o_ref[...] = jnp.dot(
            x1, w_ref[...], preferred_element_type=jnp.float32) * dinv_ref[...]

    return pl.pallas_call(
        body, out_shape=jax.ShapeDtypeStruct((N, D), jnp.float32))(
            acc, g, dinv, b, w)


def _tc_post_final(acc, g, dinv, b):
    def body(acc_ref, g_ref, dinv_ref, b_ref, o_ref):
        agg = acc_ref[0, :N, :] + acc_ref[1, :N, :] + g_ref[...]
        o_ref[...] = jnp.maximum(dinv_ref[...] * agg + b_ref[...], 0.0)

    return pl.pallas_call(
        body, out_shape=jax.ShapeDtypeStruct((N, D), jnp.float32))(
            acc, g, dinv, b)


@jax.jit
def kernel(x, edge_index, edge_weight, W1, b1, W2, b2):
    src = edge_index[0].astype(jnp.int32)
    dst = edge_index[1].astype(jnp.int32)
    ew = edge_weight.astype(jnp.float32)
    pad = EPAD - E
    src_p = jnp.pad(src, (0, pad))
    dst_p = jnp.pad(dst, (0, pad))
    ew_p = jnp.pad(ew, (0, pad))
    src4 = src_p.reshape(TOTG, G, K)
    dst4 = dst_p.reshape(TOTG, G, K)
    ew4 = ew_p.reshape(TOTG, G * K)
    dst3 = dst_p.reshape(NW, CW, K)
    ew3 = ew_p.reshape(NW, CW, K)
    zdeg = jnp.zeros((NPAD,), jnp.float32)
    zrow = jnp.zeros((NPAD, D), jnp.float32)

    deg2 = _sc_deg(dst3, ew3, zdeg)
    dinv_row = _tc_dinv(deg2)
    dinv_col = dinv_row[0, :N][:, None]

    g1 = _tc_mm_scale(x, W1, dinv_col)
    acc1 = _sc_edge(g1, src4, dst4, ew4, zrow)
    g2 = _tc_post_mm(acc1, g1, dinv_col, b1.reshape(1, D), W2)
    acc2 = _sc_edge(g2, src4, dst4, ew4, zrow)
    return _tc_post_final(acc2, g2, dinv_col, b2.reshape(1, D))


# speedup vs baseline: 1.2232x; 1.2232x over previous
import functools

import jax
import jax.numpy as jnp
from jax import lax
from jax.experimental import pallas as pl
from jax.experimental.pallas import tpu as pltpu
from jax.experimental.pallas import tpu_sc as plsc

N = 10000
E = 320000
D = 128
NC = 2
NS = 16
L = 16
NW = NC * NS
K = 128
G = 16
NG0 = 7
NG1 = 3
NG = NG0 + NG1
CW = G * NG // 2
TOTG = NS * NG
EPAD = TOTG * G * K
NPAD = 10240
RPS = NPAD // NS

_mesh = plsc.VectorSubcoreMesh(core_axis_name="c", subcore_axis_name="s")
_sc_params = pltpu.CompilerParams(needs_layout_passes=False)


@functools.partial(
    pl.kernel,
    out_type=jax.ShapeDtypeStruct((NC, NPAD), jnp.float32),
    mesh=_mesh,
    compiler_params=_sc_params,
    scratch_types=[
        pltpu.VMEM((CW, K), jnp.int32),
        pltpu.VMEM((CW, K), jnp.float32),
        pltpu.VMEM_SHARED((NPAD,), jnp.float32),
    ],
)
def _sc_deg(dst_hbm, ew_hbm, zdeg_hbm, deg_out, dst_v, ew_v, deg_sh):
    c = lax.axis_index("c")
    s = lax.axis_index("s")
    wid = s * NC + c
    pltpu.sync_copy(zdeg_hbm.at[pl.ds(s * RPS, RPS)],
                    deg_sh.at[pl.ds(s * RPS, RPS)])
    pltpu.sync_copy(dst_hbm.at[wid], dst_v)
    pltpu.sync_copy(ew_hbm.at[wid], ew_v)
    plsc.subcore_barrier()

    @pl.loop(0, CW)
    def _(t):
        pltpu.sync_copy(ew_v.at[t], deg_sh.at[dst_v.at[t]], add=True)

    plsc.subcore_barrier()

    @pl.when(s == 0)
    def _():
        pltpu.sync_copy(deg_sh, deg_out.at[c])


@functools.partial(
    pl.kernel,
    out_type=jax.ShapeDtypeStruct((NC, NPAD, D), jnp.float32),
    mesh=_mesh,
    compiler_params=_sc_params,
    scratch_types=[
        pltpu.VMEM((G, K), jnp.int32),
        pltpu.VMEM((G, K), jnp.int32),
        pltpu.VMEM((G * K,), jnp.float32),
        pltpu.VMEM((K, D), jnp.float32),
        pltpu.VMEM((K, D), jnp.float32),
        pltpu.VMEM_SHARED((NPAD, D), jnp.float32),
        pltpu.SemaphoreType.DMA,
        pltpu.SemaphoreType.DMA,
    ],
)
def _sc_edge(g_hbm, src_hbm, dst_hbm, ew_hbm, zrow_hbm, acc_out,
             src_v, dst_v, ew_v, rows_a, rows_b, acc_sh, sem_a, sem_b):
    c = lax.axis_index("c")
    s = lax.axis_index("s")
    ng = jnp.where(c == 0, NG0, NG1)
    gb = jnp.where(c == 0, s * NG0, NS * NG0 + s * NG1)
    pltpu.sync_copy(zrow_hbm.at[pl.ds(s * RPS, RPS)],
                    acc_sh.at[pl.ds(s * RPS, RPS)])
    plsc.subcore_barrier()

    def start_gather(t, rows, sem):
        pltpu.make_async_copy(g_hbm.at[src_v.at[t]], rows, sem).start()

    def wait_gather(t, rows, sem):
        pltpu.make_async_copy(g_hbm.at[src_v.at[t]], rows, sem).wait()

    def scale(t, rows):
        @pl.loop(0, K)
        def _(k):
            w = plsc.load_gather(ew_v, [jnp.full((L,), t * K + k, jnp.int32)])

            @pl.loop(0, D, step=L)
            def _(j):
                rows[k, pl.ds(j, L)] = rows[k, pl.ds(j, L)] * w

    def scatter_add(t, rows):
        pltpu.sync_copy(rows, acc_sh.at[dst_v.at[t]], add=True)

    @pl.loop(0, NG)
    def _(gi):
      @pl.when(gi < ng)
      def _():
        pltpu.sync_copy(src_hbm.at[gb + gi], src_v)
        pltpu.sync_copy(dst_hbm.at[gb + gi], dst_v)
        pltpu.sync_copy(ew_hbm.at[gb + gi], ew_v)
        start_gather(0, rows_a, sem_a)

        @pl.loop(0, G // 2)
        def _(u):
            ta = 2 * u
            tb = 2 * u + 1
            start_gather(tb, rows_b, sem_b)
            wait_gather(ta, rows_a, sem_a)
            scale(ta, rows_a)
            scatter_add(ta, rows_a)

            @pl.when(u + 1 < G // 2)
            def _():
                start_gather(ta + 2, rows_a, sem_a)

            wait_gather(tb, rows_b, sem_b)
            scale(tb, rows_b)
            scatter_add(tb, rows_b)

    plsc.subcore_barrier()
    pltpu.sync_copy(acc_sh.at[pl.ds(s * RPS, RPS)],
                    acc_out.at[c, pl.ds(s * RPS, RPS)])


def _tc_dinv(deg2):
    def body(deg_ref, out_ref):
        d = deg_ref[0:1, :] + deg_ref[1:2, :] + 1.0
        out_ref[...] = jnp.where(d > 0, lax.rsqrt(d), 0.0)

    return pl.pallas_call(
        body, out_shape=jax.ShapeDtypeStruct((1, NPAD), jnp.float32))(deg2)


def _tc_mm_scale(x, w, dinv):
    def body(x_ref, w_ref, dinv_ref, o_ref):
        h = jnp.dot(x_ref[...], w_ref[...], preferred_element_type=jnp.float32)
        o_ref[...] = h * dinv_ref[...]

    return pl.pallas_call(
        body, out_shape=jax.ShapeDtypeStruct((N, D), jnp.float32))(x, w, dinv)


def _tc_post_mm(acc, g, dinv, b, w):
    def body(acc_ref, g_ref, dinv_ref, b_ref, w_ref, o_ref):
        agg = acc_ref[0, :N, :] + acc_ref[1, :N, :] + g_ref[...]
        x1 = jnp.maximum(dinv_ref[...] * agg + b_ref[...], 0.0)
        o_ref[...] = jnp.dot(
            x1, w_ref[...], preferred_element_type=jnp.float32) * dinv_ref[...]

    return pl.pallas_call(
        body, out_shape=jax.ShapeDtypeStruct((N, D), jnp.float32))(
            acc, g, dinv, b, w)


def _tc_post_final(acc, g, dinv, b):
    def body(acc_ref, g_ref, dinv_ref, b_ref, o_ref):
        agg = acc_ref[0, :N, :] + acc_ref[1, :N, :] + g_ref[...]
        o_ref[...] = jnp.maximum(dinv_ref[...] * agg + b_ref[...], 0.0)

    return pl.pallas_call(
        body, out_shape=jax.ShapeDtypeStruct((N, D), jnp.float32))(
            acc, g, dinv, b)


@jax.jit
def kernel(x, edge_index, edge_weight, W1, b1, W2, b2):
    src = edge_index[0].astype(jnp.int32)
    dst = edge_index[1].astype(jnp.int32)
    ew = edge_weight.astype(jnp.float32)
    pad = EPAD - E
    src_p = jnp.pad(src, (0, pad))
    dst_p = jnp.pad(dst, (0, pad))
    ew_p = jnp.pad(ew, (0, pad))
    src4 = src_p.reshape(TOTG, G, K)
    dst4 = dst_p.reshape(TOTG, G, K)
    ew4 = ew_p.reshape(TOTG, G * K)
    dst3 = dst_p.reshape(NW, CW, K)
    ew3 = ew_p.reshape(NW, CW, K)
    zdeg = jnp.zeros((NPAD,), jnp.float32)
    zrow = jnp.zeros((NPAD, D), jnp.float32)

    deg2 = _sc_deg(dst3, ew3, zdeg)
    dinv_row = _tc_dinv(deg2)
    dinv_col = dinv_row[0, :N][:, None]

    g1 = _tc_mm_scale(x, W1, dinv_col)
    acc1 = _sc_edge(g1, src4, dst4, ew4, zrow)
    g2 = _tc_post_mm(acc1, g1, dinv_col, b1.reshape(1, D), W2)
    acc2 = _sc_edge(g2, src4, dst4, ew4, zrow)
    return _tc_post_final(acc2, g2, dinv_col, b2.reshape(1, D))
